# baseline (speedup 1.0000x reference)
import functools

import jax
import jax.numpy as jnp
from jax import lax
from jax.experimental import pallas as pl
from jax.experimental.pallas import tpu as pltpu
from jax.experimental.pallas import tpu_sc as plsc

B = 16384
F = 64
NC = 2
NS = 16
NW = NC * NS
BPW = B // NW
L = 16


def _body(uid, iid, text, utab, itab, ubias, ibias, out,
          uidx_v, iidx_v, urows, irows, trows, ub_v, ib_v, out_v, sem):
    wid = lax.axis_index("s") * NC + lax.axis_index("c")
    base = wid * BPW

    pltpu.sync_copy(uid.at[pl.ds(base, BPW)], uidx_v)
    pltpu.sync_copy(iid.at[pl.ds(base, BPW)], iidx_v)

    c1 = pltpu.async_copy(utab.at[uidx_v], urows, sem)
    c2 = pltpu.async_copy(itab.at[iidx_v], irows, sem)
    c3 = pltpu.async_copy(ubias.at[uidx_v], ub_v, sem)
    c4 = pltpu.async_copy(ibias.at[iidx_v], ib_v, sem)
    c5 = pltpu.async_copy(text.at[pl.ds(base, BPW)], trows, sem)
    c1.wait(); c2.wait(); c3.wait(); c4.wait(); c5.wait()

    def group(g, _):
        rb = g * L
        ridx = rb + lax.iota(jnp.int32, L)
        acc = ub_v[pl.ds(rb, L)] + ib_v[pl.ds(rb, L)]

        def col(f, acc):
            cidx = jnp.full((L,), f, jnp.int32)
            u = plsc.load_gather(urows, [ridx, cidx])
            i = plsc.load_gather(irows, [ridx, cidx])
            t = plsc.load_gather(trows, [ridx, cidx])
            return acc + u * (i + t)

        acc = lax.fori_loop(0, F, col, acc, unroll=8)
        out_v[pl.ds(rb, L)] = acc
        return 0

    lax.fori_loop(0, BPW // L, group, 0)
    pltpu.sync_copy(out_v, out.at[pl.ds(base, BPW)])


def kernel(user_id, item_id, text_embeddings, user_table, item_table,
           user_bias, item_bias):
    mesh = plsc.VectorSubcoreMesh(core_axis_name="c", subcore_axis_name="s")
    k = functools.partial(
        pl.kernel,
        out_type=jax.ShapeDtypeStruct((B,), jnp.float32),
        mesh=mesh,
        compiler_params=pltpu.CompilerParams(
            needs_layout_passes=False, use_tc_tiling_on_sc=False),
        scratch_types=[
            pltpu.VMEM((BPW,), jnp.int32),
            pltpu.VMEM((BPW,), jnp.int32),
            pltpu.VMEM((BPW, F), jnp.float32),
            pltpu.VMEM((BPW, F), jnp.float32),
            pltpu.VMEM((BPW, F), jnp.float32),
            pltpu.VMEM((BPW,), jnp.float32),
            pltpu.VMEM((BPW,), jnp.float32),
            pltpu.VMEM((BPW,), jnp.float32),
            pltpu.SemaphoreType.DMA,
        ],
    )(_body)
    out = k(user_id.reshape(B), item_id.reshape(B), text_embeddings,
            user_table, item_table,
            user_bias.reshape(user_bias.shape[0]),
            item_bias.reshape(item_bias.shape[0]))
    return out.reshape(B, 1)
